# initial kernel scaffold (unmeasured)
import jax
import jax.numpy as jnp
from jax import lax
from jax.experimental import pallas as pl
from jax.experimental.pallas import tpu as pltpu


def kernel(
    x,
):
    def body(*refs):
        pass

    out_shape = jax.ShapeDtypeStruct(..., jnp.float32)
    return pl.pallas_call(body, out_shape=out_shape)(...)


# baseline (device time: 9999 ns/iter reference)
import jax
import jax.numpy as jnp
from jax import lax
from jax.experimental import pallas as pl
from jax.experimental.pallas import tpu as pltpu

N_DEV = 4


def kernel(x):
    m_per, n = x.shape

    def body(x_ref, out_ref, comm_ref, send_sems, recv_sems):
        my_pos = lax.axis_index("i")
        left = (my_pos + N_DEV - 1) % N_DEV
        right = (my_pos + 1) % N_DEV

        barrier_sem = pltpu.get_barrier_semaphore()
        for nbr in [left, right]:
            pl.semaphore_signal(
                barrier_sem, inc=1,
                device_id=(nbr,), device_id_type=pl.DeviceIdType.MESH,
            )
        pl.semaphore_wait(barrier_sem, 2)

        x_val = x_ref[:, :]

        comm_ref[0, :, :] = jnp.sum(x_val, axis=0, keepdims=True)

        r = lax.broadcasted_iota(jnp.int32, (m_per, m_per), 0)
        c = lax.broadcasted_iota(jnp.int32, (m_per, m_per), 1)
        tri = (r >= c).astype(jnp.float32)
        local = jnp.dot(tri, x_val, preferred_element_type=jnp.float32)

        offset = jnp.zeros((1, n), jnp.float32)
        for h in range(N_DEV - 1):
            send_slot = h % 2
            recv_slot = (h + 1) % 2
            rdma = pltpu.make_async_remote_copy(
                src_ref=comm_ref.at[send_slot],
                dst_ref=comm_ref.at[recv_slot],
                send_sem=send_sems.at[send_slot],
                recv_sem=recv_sems.at[recv_slot],
                device_id=(right,),
                device_id_type=pl.DeviceIdType.MESH,
            )
            rdma.start()
            rdma.wait()

            origin = (my_pos + N_DEV - h - 1) % N_DEV
            chunk = comm_ref[recv_slot, :, :]
            offset = offset + jnp.where(origin < my_pos, chunk, 0.0)

        out_ref[:, :] = local + offset

    return pl.pallas_call(
        body,
        out_shape=jax.ShapeDtypeStruct((m_per, n), jnp.float32),
        in_specs=[pl.BlockSpec(memory_space=pltpu.VMEM)],
        out_specs=pl.BlockSpec(memory_space=pltpu.VMEM),
        scratch_shapes=[
            pltpu.VMEM((2, 1, n), jnp.float32),
            pltpu.SemaphoreType.DMA((2,)),
            pltpu.SemaphoreType.DMA((2,)),
        ],
        compiler_params=pltpu.CompilerParams(collective_id=0),
    )(x)


# device time: 8627 ns/iter; 1.1590x vs baseline; 1.1590x over previous
import jax
import jax.numpy as jnp
from jax import lax
from jax.experimental import pallas as pl
from jax.experimental.pallas import tpu as pltpu

N_DEV = 4


def kernel(x):
    m_per, n = x.shape

    def body(x_ref, out_ref, send_buf, gather_ref, send_sems, recv_sems,
             credit_sem):
        my_pos = lax.axis_index("i")

        barrier_sem = pltpu.get_barrier_semaphore()
        for k in range(1, N_DEV):
            pl.semaphore_signal(
                barrier_sem, inc=1,
                device_id=((my_pos + k) % N_DEV,),
                device_id_type=pl.DeviceIdType.MESH,
            )
        pl.semaphore_wait(barrier_sem, N_DEV - 1)

        x_val = x_ref[:, :]
        send_buf[:, :] = jnp.sum(x_val, axis=0, keepdims=True)

        def make_send(s, d):
            return pltpu.make_async_remote_copy(
                src_ref=send_buf,
                dst_ref=gather_ref.at[s],
                send_sem=send_sems.at[d - s - 1],
                recv_sem=recv_sems.at[s],
                device_id=(d,),
                device_id_type=pl.DeviceIdType.MESH,
            )

        for s in range(N_DEV - 1):
            @pl.when(my_pos == s)
            def _(s=s):
                for d in range(s + 1, N_DEV):
                    make_send(s, d).start()

        r = lax.broadcasted_iota(jnp.int32, (m_per, m_per), 0)
        c = lax.broadcasted_iota(jnp.int32, (m_per, m_per), 1)
        tri = (r >= c).astype(jnp.float32)
        local = jnp.dot(tri, x_val, preferred_element_type=jnp.float32)

        for s in range(N_DEV - 1):
            @pl.when(my_pos > s)
            def _(s=s):
                make_send(s, (s + 1) % N_DEV).wait_recv()

        offset = jnp.zeros((1, n), jnp.float32)
        for s in range(N_DEV - 1):
            offset = offset + jnp.where(
                my_pos > s, gather_ref[s, :, :], 0.0)

        for s in range(N_DEV - 1):
            @pl.when(my_pos > s)
            def _(s=s):
                pl.semaphore_signal(
                    credit_sem, inc=1,
                    device_id=(s,), device_id_type=pl.DeviceIdType.MESH,
                )

        out_ref[:, :] = local + offset

        for s in range(N_DEV - 1):
            @pl.when(my_pos == s)
            def _(s=s):
                for d in range(s + 1, N_DEV):
                    make_send(s, d).wait_send()
                pl.semaphore_wait(credit_sem, N_DEV - 1 - s)

    return pl.pallas_call(
        body,
        out_shape=jax.ShapeDtypeStruct((m_per, n), jnp.float32),
        in_specs=[pl.BlockSpec(memory_space=pltpu.VMEM)],
        out_specs=pl.BlockSpec(memory_space=pltpu.VMEM),
        scratch_shapes=[
            pltpu.VMEM((1, n), jnp.float32),
            pltpu.VMEM((N_DEV - 1, 1, n), jnp.float32),
            pltpu.SemaphoreType.DMA((N_DEV - 1,)),
            pltpu.SemaphoreType.DMA((N_DEV - 1,)),
            pltpu.SemaphoreType.REGULAR,
        ],
        compiler_params=pltpu.CompilerParams(collective_id=0),
    )(x)
